# baseline (speedup 1.0000x reference)
import functools

import jax
import jax.numpy as jnp
from jax import lax
from jax.experimental import pallas as pl
from jax.experimental.pallas import tpu as pltpu

B, A, M, C = 8, 20000, 100, 80
A_PAD = 20480
BLK = 4096
NBLK = A_PAD // BLK
R = BLK // 128


def _focal_kernel(tbl_ref, anc_ref, cls_ref, reg_ref, out_c_ref, out_r_ref,
                  pbuf, cbuf, acc_c, acc_r, acc_n, oc_acc, or_acc):
    b = pl.program_id(0)
    i = pl.program_id(1)

    @pl.when(i < NBLK)
    def _scan_and_stage():
        anc = anc_ref[0]
        a_y1 = anc[0 * R:1 * R]
        a_x1 = anc[1 * R:2 * R]
        a_y2 = anc[2 * R:3 * R]
        a_x2 = anc[3 * R:4 * R]
        area_a = (a_x2 - a_x1) * (a_y2 - a_y1)

        zero = jnp.zeros((R, 128), jnp.float32)

        def body(m, carry):
            run_max, ax1, ay1, ax2, ay2, lab = carry
            bx1 = tbl_ref[b, 0, m]
            by1 = tbl_ref[b, 1, m]
            bx2 = tbl_ref[b, 2, m]
            by2 = tbl_ref[b, 3, m]
            barea = tbl_ref[b, 4, m]
            blab = tbl_ref[b, 5, m]
            iw = jnp.maximum(
                jnp.minimum(a_x2, bx2) - jnp.maximum(a_x1, bx1), 0.0)
            ih = jnp.maximum(
                jnp.minimum(a_y2, by2) - jnp.maximum(a_y1, by1), 0.0)
            inter = iw * ih
            ua = area_a + barea - inter
            iou = inter / ua
            take = iou > run_max
            return (jnp.maximum(run_max, iou),
                    jnp.where(take, bx1, ax1),
                    jnp.where(take, by1, ay1),
                    jnp.where(take, bx2, ax2),
                    jnp.where(take, by2, ay2),
                    jnp.where(take, blab, lab))

        carry = (jnp.full((R, 128), -jnp.inf, jnp.float32),
                 zero, zero, zero, zero, zero)
        for m in range(M):
            carry = body(m, carry)
        iou_max, ab_x1, ab_y1, ab_x2, ab_y2, alab = carry

        gt_w_raw = ab_x2 - ab_x1
        gt_h_raw = ab_y2 - ab_y1
        thr = jnp.where(gt_w_raw * gt_h_raw > 100.0, 0.5, 0.15)
        pos = iou_max >= thr
        posf = jnp.where(pos, 1.0, 0.0)
        npos_part = jnp.sum(posf, keepdims=True)

        buf = i % 2
        pbuf[buf] = cls_ref[0].T
        cbuf[buf] = jnp.where(pos, alab, -1.0)[None]

        aw0 = a_x2 - a_x1
        ah0 = a_y2 - a_y1
        ctr_x = a_x1 + 0.5 * aw0
        ctr_y = a_y1 + 0.5 * ah0
        aw = jnp.where(pos, aw0, 1.0)
        ah = jnp.where(pos, ah0, 1.0)
        gt_cx = ab_x1 + 0.5 * gt_w_raw
        gt_cy = ab_y1 + 0.5 * gt_h_raw
        gt_w = jnp.maximum(gt_w_raw, 1.0)
        gt_h = jnp.maximum(gt_h_raw, 1.0)
        tdx = (gt_cx - ctr_x) / aw
        tdy = (gt_cy - ctr_y) / ah
        tdw = jnp.log(gt_w / aw)
        tdh = jnp.log(gt_h / ah)

        reg4 = reg_ref[0, 0]
        r_dy = reg4[0 * R:1 * R]
        r_dx = reg4[1 * R:2 * R]
        r_dh = reg4[2 * R:3 * R]
        r_dw = reg4[3 * R:4 * R]

        def smooth(d):
            return jnp.where(d <= 1.0 / 9.0, 4.5 * d * d, d - 1.0 / 18.0)

        rsum = (smooth(jnp.abs(tdy - r_dy)) + smooth(jnp.abs(tdx - r_dx))
                + smooth(jnp.abs(tdh - r_dh)) + smooth(jnp.abs(tdw - r_dw)))
        reg_part = jnp.sum(rsum * posf, keepdims=True)

        first = i == 0
        r0 = jnp.where(first, 0.0, acc_r[0:1, 0:1]) + reg_part
        n0 = jnp.where(first, 0.0, acc_n[0:1, 0:1]) + npos_part
        acc_r[0:1, 0:1] = r0
        acc_n[0:1, 0:1] = n0

    @pl.when(i > 0)
    def _focal_pass():
        buf = (i - 1) % 2
        p = jnp.clip(pbuf[buf], 1e-4, 1.0 - 1e-4)
        neg = (0.75 * (p * p)) * (-jnp.log(1.0 - p))
        col = lax.broadcasted_iota(jnp.int32, (1, BLK), 1) + (i - 1) * BLK
        neg_cols = jnp.sum(neg, axis=0, keepdims=True)
        neg_sum = jnp.sum(jnp.where(col < A, neg_cols, 0.0), keepdims=True)
        code_row = cbuf[buf].reshape(1, BLK)
        c_iota = lax.broadcasted_iota(
            jnp.int32, (C, 1), 0).astype(jnp.float32)
        sel = c_iota == code_row
        p_sel = jnp.sum(jnp.where(sel, p, 0.0), axis=0, keepdims=True)
        p_c = jnp.clip(p_sel, 1e-4, 1.0)
        g = (0.25 * (1.0 - p_c) * (1.0 - p_c)) * (-jnp.log(p_c)) \
            - (0.75 * (p_c * p_c)) * (-jnp.log(1.0 - p_c))
        corr = jnp.where(code_row >= 0.0, g, 0.0)
        cls_part = neg_sum + jnp.sum(corr, keepdims=True)
        c0 = jnp.where(i == 1, 0.0, acc_c[0:1, 0:1]) + cls_part
        acc_c[0:1, 0:1] = c0

    @pl.when(i == NBLK)
    def _finish_batch():
        c0 = acc_c[0:1, 0:1]
        r0 = acc_r[0:1, 0:1]
        n0 = acc_n[0:1, 0:1]
        den = jnp.maximum(n0, 1.0)
        cb = c0 / den
        rb = jnp.where(n0 > 0.0, r0 / (4.0 * den), 0.0) * 50.0
        oc = jnp.where(b == 0, 0.0, oc_acc[0:1, 0:1]) + cb
        orr = jnp.where(b == 0, 0.0, or_acc[0:1, 0:1]) + rb
        oc_acc[0:1, 0:1] = oc
        or_acc[0:1, 0:1] = orr
        out_c_ref[0:1, 0:1] = oc * (1.0 / B)
        out_r_ref[0:1, 0:1] = orr * (1.0 / B)


@functools.partial(jax.jit)
def _run(tbl, anc_pack, cls, reg_pack):
    clamp = lambda i: jnp.minimum(i, NBLK - 1)
    out_c, out_r = pl.pallas_call(
        _focal_kernel,
        grid=(B, NBLK + 1),
        in_specs=[
            pl.BlockSpec(memory_space=pltpu.SMEM),
            pl.BlockSpec((1, 4 * R, 128), lambda b, i: (clamp(i), 0, 0)),
            pl.BlockSpec((1, BLK, C), lambda b, i: (b, clamp(i), 0)),
            pl.BlockSpec((1, 1, 4 * R, 128),
                         lambda b, i: (b, clamp(i), 0, 0)),
        ],
        out_specs=[
            pl.BlockSpec((1, 1), lambda b, i: (0, 0)),
            pl.BlockSpec((1, 1), lambda b, i: (0, 0)),
        ],
        out_shape=[
            jax.ShapeDtypeStruct((1, 1), jnp.float32),
            jax.ShapeDtypeStruct((1, 1), jnp.float32),
        ],
        scratch_shapes=[pltpu.VMEM((2, C, BLK), jnp.float32),
                        pltpu.VMEM((2, 1, R, 128), jnp.float32)]
                       + [pltpu.VMEM((1, 1), jnp.float32)] * 5,
        compiler_params=pltpu.CompilerParams(
            dimension_semantics=("arbitrary", "arbitrary")),
    )(tbl, anc_pack, cls, reg_pack)
    return out_c.reshape(1), out_r.reshape(1)


def kernel(detection_boxes, detection_labels, anchors, classification, regression):
    valid = detection_labels != 0
    bx = jnp.where(valid[:, :, None], detection_boxes, 1e9)
    labf = (detection_labels - 1).astype(jnp.float32)
    area_b = jnp.where(valid,
                       (bx[..., 2] - bx[..., 0]) * (bx[..., 3] - bx[..., 1]),
                       0.0)
    tbl = jnp.stack([bx[..., 0], bx[..., 1], bx[..., 2], bx[..., 3],
                     area_b, labf], axis=1)

    pad_anc = jnp.broadcast_to(
        jnp.array([-10.0, -10.0, -9.0, -9.0], jnp.float32),
        (A_PAD - A, 4))
    anc0 = jnp.concatenate([anchors[0], pad_anc], axis=0)
    anc_pack = (anc0.T.reshape(4, NBLK, R, 128)
                .transpose(1, 0, 2, 3).reshape(NBLK, 4 * R, 128))

    regp = jnp.pad(regression, ((0, 0), (0, A_PAD - A), (0, 0)))
    reg_pack = (regp.transpose(0, 2, 1).reshape(B, 4, NBLK, R, 128)
                .transpose(0, 2, 1, 3, 4).reshape(B, NBLK, 4 * R, 128))

    return _run(tbl, anc_pack, classification, reg_pack)


# speedup vs baseline: 1.0198x; 1.0198x over previous
import functools

import jax
import jax.numpy as jnp
from jax import lax
from jax.experimental import pallas as pl
from jax.experimental.pallas import tpu as pltpu

B, A, M, C = 8, 20000, 100, 80
A_PAD = 20480
BLK = 4096
NBLK = A_PAD // BLK
R = BLK // 128


def _focal_kernel(tbl_ref, anc_ref, cls_ref, reg_ref, out_c_ref, out_r_ref,
                  pbuf, cbuf, acc_c, acc_r, acc_n, oc_acc, or_acc):
    b = pl.program_id(0)
    i = pl.program_id(1)

    @pl.when(i < NBLK)
    def _scan_and_stage():
        anc = anc_ref[0]
        a_y1 = anc[0 * R:1 * R]
        a_x1 = anc[1 * R:2 * R]
        a_y2 = anc[2 * R:3 * R]
        a_x2 = anc[3 * R:4 * R]
        area_a = (a_x2 - a_x1) * (a_y2 - a_y1)

        zero = jnp.zeros((R, 128), jnp.float32)

        def body(m, carry):
            run_max, ax1, ay1, ax2, ay2, lab = carry
            bx1 = tbl_ref[b, 0, m]
            by1 = tbl_ref[b, 1, m]
            bx2 = tbl_ref[b, 2, m]
            by2 = tbl_ref[b, 3, m]
            barea = tbl_ref[b, 4, m]
            blab = tbl_ref[b, 5, m]
            iw = jnp.maximum(
                jnp.minimum(a_x2, bx2) - jnp.maximum(a_x1, bx1), 0.0)
            ih = jnp.maximum(
                jnp.minimum(a_y2, by2) - jnp.maximum(a_y1, by1), 0.0)
            inter = iw * ih
            ua = area_a + barea - inter
            iou = inter / ua
            take = iou > run_max
            return (jnp.maximum(run_max, iou),
                    jnp.where(take, bx1, ax1),
                    jnp.where(take, by1, ay1),
                    jnp.where(take, bx2, ax2),
                    jnp.where(take, by2, ay2),
                    jnp.where(take, blab, lab))

        carry = (jnp.full((R, 128), -jnp.inf, jnp.float32),
                 zero, zero, zero, zero, zero)
        for m in range(M):
            carry = body(m, carry)
        iou_max, ab_x1, ab_y1, ab_x2, ab_y2, alab = carry

        gt_w_raw = ab_x2 - ab_x1
        gt_h_raw = ab_y2 - ab_y1
        thr = jnp.where(gt_w_raw * gt_h_raw > 100.0, 0.5, 0.15)
        pos = iou_max >= thr
        posf = jnp.where(pos, 1.0, 0.0)
        npos_part = jnp.sum(posf, keepdims=True)

        buf = i % 2
        ident = (lax.broadcasted_iota(jnp.int32, (128, 128), 0)
                 == lax.broadcasted_iota(jnp.int32, (128, 128), 1)
                 ).astype(jnp.float32)
        for k in range(BLK // 128):
            chunk = cls_ref[0][k * 128:(k + 1) * 128, :]
            pbuf[buf, :, k * 128:(k + 1) * 128] = lax.dot_general(
                chunk, ident, (((0,), (0,)), ((), ())),
                preferred_element_type=jnp.float32)
        cbuf[buf] = jnp.where(pos, alab, -1.0)[None]

        aw0 = a_x2 - a_x1
        ah0 = a_y2 - a_y1
        ctr_x = a_x1 + 0.5 * aw0
        ctr_y = a_y1 + 0.5 * ah0
        aw = jnp.where(pos, aw0, 1.0)
        ah = jnp.where(pos, ah0, 1.0)
        gt_cx = ab_x1 + 0.5 * gt_w_raw
        gt_cy = ab_y1 + 0.5 * gt_h_raw
        gt_w = jnp.maximum(gt_w_raw, 1.0)
        gt_h = jnp.maximum(gt_h_raw, 1.0)
        tdx = (gt_cx - ctr_x) / aw
        tdy = (gt_cy - ctr_y) / ah
        tdw = jnp.log(gt_w / aw)
        tdh = jnp.log(gt_h / ah)

        reg4 = reg_ref[0, 0]
        r_dy = reg4[0 * R:1 * R]
        r_dx = reg4[1 * R:2 * R]
        r_dh = reg4[2 * R:3 * R]
        r_dw = reg4[3 * R:4 * R]

        def smooth(d):
            return jnp.where(d <= 1.0 / 9.0, 4.5 * d * d, d - 1.0 / 18.0)

        rsum = (smooth(jnp.abs(tdy - r_dy)) + smooth(jnp.abs(tdx - r_dx))
                + smooth(jnp.abs(tdh - r_dh)) + smooth(jnp.abs(tdw - r_dw)))
        reg_part = jnp.sum(rsum * posf, keepdims=True)

        first = i == 0
        r0 = jnp.where(first, 0.0, acc_r[0:1, 0:1]) + reg_part
        n0 = jnp.where(first, 0.0, acc_n[0:1, 0:1]) + npos_part
        acc_r[0:1, 0:1] = r0
        acc_n[0:1, 0:1] = n0

    @pl.when(i > 0)
    def _focal_pass():
        buf = (i - 1) % 2
        p = jnp.clip(pbuf[buf], 1e-4, 1.0 - 1e-4)
        neg = (0.75 * (p * p)) * (-jnp.log(1.0 - p))
        col = lax.broadcasted_iota(jnp.int32, (1, BLK), 1) + (i - 1) * BLK
        neg_cols = jnp.sum(neg, axis=0, keepdims=True)
        neg_sum = jnp.sum(jnp.where(col < A, neg_cols, 0.0), keepdims=True)
        code_row = cbuf[buf].reshape(1, BLK)
        c_iota = lax.broadcasted_iota(
            jnp.int32, (C, 1), 0).astype(jnp.float32)
        sel = c_iota == code_row
        p_sel = jnp.sum(jnp.where(sel, p, 0.0), axis=0, keepdims=True)
        p_c = jnp.clip(p_sel, 1e-4, 1.0)
        g = (0.25 * (1.0 - p_c) * (1.0 - p_c)) * (-jnp.log(p_c)) \
            - (0.75 * (p_c * p_c)) * (-jnp.log(1.0 - p_c))
        corr = jnp.where(code_row >= 0.0, g, 0.0)
        cls_part = neg_sum + jnp.sum(corr, keepdims=True)
        c0 = jnp.where(i == 1, 0.0, acc_c[0:1, 0:1]) + cls_part
        acc_c[0:1, 0:1] = c0

    @pl.when(i == NBLK)
    def _finish_batch():
        c0 = acc_c[0:1, 0:1]
        r0 = acc_r[0:1, 0:1]
        n0 = acc_n[0:1, 0:1]
        den = jnp.maximum(n0, 1.0)
        cb = c0 / den
        rb = jnp.where(n0 > 0.0, r0 / (4.0 * den), 0.0) * 50.0
        oc = jnp.where(b == 0, 0.0, oc_acc[0:1, 0:1]) + cb
        orr = jnp.where(b == 0, 0.0, or_acc[0:1, 0:1]) + rb
        oc_acc[0:1, 0:1] = oc
        or_acc[0:1, 0:1] = orr
        out_c_ref[0:1, 0:1] = oc * (1.0 / B)
        out_r_ref[0:1, 0:1] = orr * (1.0 / B)


@functools.partial(jax.jit)
def _run(tbl, anc_pack, cls, reg_pack):
    clamp = lambda i: jnp.minimum(i, NBLK - 1)
    out_c, out_r = pl.pallas_call(
        _focal_kernel,
        grid=(B, NBLK + 1),
        in_specs=[
            pl.BlockSpec(memory_space=pltpu.SMEM),
            pl.BlockSpec((1, 4 * R, 128), lambda b, i: (clamp(i), 0, 0)),
            pl.BlockSpec((1, BLK, C), lambda b, i: (b, clamp(i), 0)),
            pl.BlockSpec((1, 1, 4 * R, 128),
                         lambda b, i: (b, clamp(i), 0, 0)),
        ],
        out_specs=[
            pl.BlockSpec((1, 1), lambda b, i: (0, 0)),
            pl.BlockSpec((1, 1), lambda b, i: (0, 0)),
        ],
        out_shape=[
            jax.ShapeDtypeStruct((1, 1), jnp.float32),
            jax.ShapeDtypeStruct((1, 1), jnp.float32),
        ],
        scratch_shapes=[pltpu.VMEM((2, C, BLK), jnp.float32),
                        pltpu.VMEM((2, 1, R, 128), jnp.float32)]
                       + [pltpu.VMEM((1, 1), jnp.float32)] * 5,
        compiler_params=pltpu.CompilerParams(
            dimension_semantics=("arbitrary", "arbitrary")),
    )(tbl, anc_pack, cls, reg_pack)
    return out_c.reshape(1), out_r.reshape(1)


def kernel(detection_boxes, detection_labels, anchors, classification, regression):
    valid = detection_labels != 0
    bx = jnp.where(valid[:, :, None], detection_boxes, 1e9)
    labf = (detection_labels - 1).astype(jnp.float32)
    area_b = jnp.where(valid,
                       (bx[..., 2] - bx[..., 0]) * (bx[..., 3] - bx[..., 1]),
                       0.0)
    tbl = jnp.stack([bx[..., 0], bx[..., 1], bx[..., 2], bx[..., 3],
                     area_b, labf], axis=1)

    pad_anc = jnp.broadcast_to(
        jnp.array([-10.0, -10.0, -9.0, -9.0], jnp.float32),
        (A_PAD - A, 4))
    anc0 = jnp.concatenate([anchors[0], pad_anc], axis=0)
    anc_pack = (anc0.T.reshape(4, NBLK, R, 128)
                .transpose(1, 0, 2, 3).reshape(NBLK, 4 * R, 128))

    regp = jnp.pad(regression, ((0, 0), (0, A_PAD - A), (0, 0)))
    reg_pack = (regp.transpose(0, 2, 1).reshape(B, 4, NBLK, R, 128)
                .transpose(0, 2, 1, 3, 4).reshape(B, NBLK, 4 * R, 128))

    return _run(tbl, anc_pack, classification, reg_pack)


# speedup vs baseline: 1.4541x; 1.4258x over previous
import functools

import jax
import jax.numpy as jnp
from jax.experimental import pallas as pl
from jax.experimental.pallas import tpu as pltpu

ALPHA = 0.25

B, A, M, C = 8, 20000, 100, 80
A_PAD = 20480
BLK = 5120
NBLK = A_PAD // BLK
R = BLK // 128


def _focal_kernel(tbl_ref, anc_ref, cls_ref, reg_ref, out_c_ref, out_r_ref,
                  acc_c, acc_r, acc_n, oc_acc, or_acc):
    b = pl.program_id(0)
    i = pl.program_id(1)

    anc = anc_ref[0]
    a_y1 = anc[0 * R:1 * R]
    a_x1 = anc[1 * R:2 * R]
    a_y2 = anc[2 * R:3 * R]
    a_x2 = anc[3 * R:4 * R]
    area_a = (a_x2 - a_x1) * (a_y2 - a_y1)

    zero = jnp.zeros((R, 128), jnp.float32)

    def body(m, carry):
        run_max, ax1, ay1, ax2, ay2, lab = carry
        bx1 = tbl_ref[b, 0, m]
        by1 = tbl_ref[b, 1, m]
        bx2 = tbl_ref[b, 2, m]
        by2 = tbl_ref[b, 3, m]
        barea = tbl_ref[b, 4, m]
        blab = tbl_ref[b, 5, m]
        iw = jnp.maximum(jnp.minimum(a_x2, bx2) - jnp.maximum(a_x1, bx1), 0.0)
        ih = jnp.maximum(jnp.minimum(a_y2, by2) - jnp.maximum(a_y1, by1), 0.0)
        inter = iw * ih
        ua = area_a + barea - inter
        iou = inter / ua
        take = iou > run_max
        return (jnp.maximum(run_max, iou),
                jnp.where(take, bx1, ax1),
                jnp.where(take, by1, ay1),
                jnp.where(take, bx2, ax2),
                jnp.where(take, by2, ay2),
                jnp.where(take, blab, lab))

    carry = (jnp.full((R, 128), -jnp.inf, jnp.float32),
             zero, zero, zero, zero, zero)
    for m in range(M):
        carry = body(m, carry)
    iou_max, ab_x1, ab_y1, ab_x2, ab_y2, alab = carry

    gt_w_raw = ab_x2 - ab_x1
    gt_h_raw = ab_y2 - ab_y1
    thr = jnp.where(gt_w_raw * gt_h_raw > 100.0, 0.5, 0.15)
    pos = iou_max >= thr
    posf = jnp.where(pos, 1.0, 0.0)
    npos_part = jnp.sum(posf, keepdims=True)

    p = jnp.clip(cls_ref[0], 1e-4, 1.0 - 1e-4)
    neg = (0.75 * (p * p)) * (-jnp.log(1.0 - p))
    neg_sum = jnp.sum(neg, keepdims=True)
    code = jnp.where(pos, alab, -1.0)
    code_row = code.reshape(1, BLK)
    c_iota = jax.lax.broadcasted_iota(jnp.int32, (C, 1), 0).astype(jnp.float32)
    sel = c_iota == code_row
    p_sel = jnp.sum(jnp.where(sel, p, 0.0), axis=0, keepdims=True)
    p_c = jnp.clip(p_sel, 1e-4, 1.0)
    g = (0.25 * (1.0 - p_c) * (1.0 - p_c)) * (-jnp.log(p_c)) \
        - (0.75 * (p_c * p_c)) * (-jnp.log(1.0 - p_c))
    corr = jnp.where(code_row >= 0.0, g, 0.0)
    cls_part = neg_sum + jnp.sum(corr, keepdims=True)

    aw0 = a_x2 - a_x1
    ah0 = a_y2 - a_y1
    ctr_x = a_x1 + 0.5 * aw0
    ctr_y = a_y1 + 0.5 * ah0
    aw = jnp.where(pos, aw0, 1.0)
    ah = jnp.where(pos, ah0, 1.0)
    gt_cx = ab_x1 + 0.5 * gt_w_raw
    gt_cy = ab_y1 + 0.5 * gt_h_raw
    gt_w = jnp.maximum(gt_w_raw, 1.0)
    gt_h = jnp.maximum(gt_h_raw, 1.0)
    tdx = (gt_cx - ctr_x) / aw
    tdy = (gt_cy - ctr_y) / ah
    tdw = jnp.log(gt_w / aw)
    tdh = jnp.log(gt_h / ah)

    reg4 = reg_ref[0, 0]
    r_dy = reg4[0 * R:1 * R]
    r_dx = reg4[1 * R:2 * R]
    r_dh = reg4[2 * R:3 * R]
    r_dw = reg4[3 * R:4 * R]

    def smooth(d):
        return jnp.where(d <= 1.0 / 9.0, 4.5 * d * d, d - 1.0 / 18.0)

    rsum = (smooth(jnp.abs(tdy - r_dy)) + smooth(jnp.abs(tdx - r_dx))
            + smooth(jnp.abs(tdh - r_dh)) + smooth(jnp.abs(tdw - r_dw)))
    reg_part = jnp.sum(rsum * posf, keepdims=True)

    first = i == 0
    c0 = jnp.where(first, 0.0, acc_c[0:1, 0:1]) + cls_part
    r0 = jnp.where(first, 0.0, acc_r[0:1, 0:1]) + reg_part
    n0 = jnp.where(first, 0.0, acc_n[0:1, 0:1]) + npos_part
    acc_c[0:1, 0:1] = c0
    acc_r[0:1, 0:1] = r0
    acc_n[0:1, 0:1] = n0

    @pl.when(i == NBLK - 1)
    def _finish_batch():
        den = jnp.maximum(n0, 1.0)
        cb = c0 / den
        rb = jnp.where(n0 > 0.0, r0 / (4.0 * den), 0.0) * 50.0
        oc = jnp.where(b == 0, 0.0, oc_acc[0:1, 0:1]) + cb
        orr = jnp.where(b == 0, 0.0, or_acc[0:1, 0:1]) + rb
        oc_acc[0:1, 0:1] = oc
        or_acc[0:1, 0:1] = orr
        out_c_ref[0:1, 0:1] = oc * (1.0 / B)
        out_r_ref[0:1, 0:1] = orr * (1.0 / B)


@functools.partial(jax.jit)
def _run(tbl, anc_pack, cls, reg_pack):
    out_c, out_r = pl.pallas_call(
        _focal_kernel,
        grid=(B, NBLK),
        in_specs=[
            pl.BlockSpec(memory_space=pltpu.SMEM),
            pl.BlockSpec((1, 4 * R, 128), lambda b, i: (i, 0, 0)),
            pl.BlockSpec((1, C, BLK), lambda b, i: (b, 0, i)),
            pl.BlockSpec((1, 1, 4 * R, 128), lambda b, i: (b, i, 0, 0)),
        ],
        out_specs=[
            pl.BlockSpec((1, 1), lambda b, i: (0, 0)),
            pl.BlockSpec((1, 1), lambda b, i: (0, 0)),
        ],
        out_shape=[
            jax.ShapeDtypeStruct((1, 1), jnp.float32),
            jax.ShapeDtypeStruct((1, 1), jnp.float32),
        ],
        scratch_shapes=[pltpu.VMEM((1, 1), jnp.float32)] * 5,
        compiler_params=pltpu.CompilerParams(
            dimension_semantics=("arbitrary", "arbitrary")),
    )(tbl, anc_pack, cls, reg_pack)
    return out_c.reshape(1), out_r.reshape(1)


def kernel(detection_boxes, detection_labels, anchors, classification, regression):
    valid = detection_labels != 0
    bx = jnp.where(valid[:, :, None], detection_boxes, 1e9)
    labf = (detection_labels - 1).astype(jnp.float32)
    area_b = jnp.where(valid,
                       (bx[..., 2] - bx[..., 0]) * (bx[..., 3] - bx[..., 1]),
                       0.0)
    tbl = jnp.stack([bx[..., 0], bx[..., 1], bx[..., 2], bx[..., 3],
                     area_b, labf], axis=1)

    pad_anc = jnp.broadcast_to(
        jnp.array([-10.0, -10.0, -9.0, -9.0], jnp.float32),
        (A_PAD - A, 4))
    anc0 = jnp.concatenate([anchors[0], pad_anc], axis=0)
    anc_pack = (anc0.T.reshape(4, NBLK, R, 128)
                .transpose(1, 0, 2, 3).reshape(NBLK, 4 * R, 128))

    cls_t = jnp.pad(jnp.transpose(classification, (0, 2, 1)),
                    ((0, 0), (0, 0), (0, A_PAD - A)))

    regp = jnp.pad(regression, ((0, 0), (0, A_PAD - A), (0, 0)))
    reg_pack = (regp.transpose(0, 2, 1).reshape(B, 4, NBLK, R, 128)
                .transpose(0, 2, 1, 3, 4).reshape(B, NBLK, 4 * R, 128))

    return _run(tbl, anc_pack, cls_t, reg_pack)


# speedup vs baseline: 1.6899x; 1.1622x over previous
import functools

import jax
import jax.numpy as jnp
from jax.experimental import pallas as pl
from jax.experimental.pallas import tpu as pltpu

ALPHA = 0.25

B, A, M, C = 8, 20000, 100, 80
A_PAD = 20480
BLK = 5120
NBLK = A_PAD // BLK
R = BLK // 128


def _focal_kernel(tbl_ref, anc_ref, cls_ref, reg_ref, out_c_ref, out_r_ref,
                  acc_c, acc_r, acc_n, oc_acc, or_acc):
    b = pl.program_id(0)
    i = pl.program_id(1)

    anc = anc_ref[0]
    a_y1 = anc[0 * R:1 * R]
    a_x1 = anc[1 * R:2 * R]
    a_y2 = anc[2 * R:3 * R]
    a_x2 = anc[3 * R:4 * R]
    area_a = (a_x2 - a_x1) * (a_y2 - a_y1)

    zero = jnp.zeros((R, 128), jnp.float32)

    def body(m, carry):
        run_max, ax1, ay1, ax2, ay2, lab = carry
        bx1 = tbl_ref[b, 0, m]
        by1 = tbl_ref[b, 1, m]
        bx2 = tbl_ref[b, 2, m]
        by2 = tbl_ref[b, 3, m]
        barea = tbl_ref[b, 4, m]
        blab = tbl_ref[b, 5, m]
        iw = jnp.maximum(jnp.minimum(a_x2, bx2) - jnp.maximum(a_x1, bx1), 0.0)
        ih = jnp.maximum(jnp.minimum(a_y2, by2) - jnp.maximum(a_y1, by1), 0.0)
        inter = iw * ih
        ua = area_a + barea - inter
        iou = inter / ua
        take = iou > run_max
        return (jnp.maximum(run_max, iou),
                jnp.where(take, bx1, ax1),
                jnp.where(take, by1, ay1),
                jnp.where(take, bx2, ax2),
                jnp.where(take, by2, ay2),
                jnp.where(take, blab, lab))

    carry = (jnp.full((R, 128), -jnp.inf, jnp.float32),
             zero, zero, zero, zero, zero)
    for m in range(M):
        carry = body(m, carry)
    iou_max, ab_x1, ab_y1, ab_x2, ab_y2, alab = carry

    gt_w_raw = ab_x2 - ab_x1
    gt_h_raw = ab_y2 - ab_y1
    thr = jnp.where(gt_w_raw * gt_h_raw > 100.0, 0.5, 0.15)
    pos = iou_max >= thr
    posf = jnp.where(pos, 1.0, 0.0)
    npos_part = jnp.sum(posf, keepdims=True)

    p = jnp.clip(cls_ref[0], 1e-4, 1.0 - 1e-4)
    neg = (0.75 * (p * p)) * (-jnp.log(1.0 - p))
    col = jax.lax.broadcasted_iota(jnp.int32, (1, BLK), 1) + i * BLK
    neg_cols = jnp.sum(neg, axis=0, keepdims=True)
    neg_sum = jnp.sum(jnp.where(col < A, neg_cols, 0.0), keepdims=True)
    code = jnp.where(pos, alab, -1.0)
    code_row = code.reshape(1, BLK)
    c_iota = jax.lax.broadcasted_iota(jnp.int32, (C, 1), 0).astype(jnp.float32)
    sel = c_iota == code_row
    p_sel = jnp.sum(jnp.where(sel, p, 0.0), axis=0, keepdims=True)
    p_c = jnp.clip(p_sel, 1e-4, 1.0)
    g = (0.25 * (1.0 - p_c) * (1.0 - p_c)) * (-jnp.log(p_c)) \
        - (0.75 * (p_c * p_c)) * (-jnp.log(1.0 - p_c))
    corr = jnp.where(code_row >= 0.0, g, 0.0)
    cls_part = neg_sum + jnp.sum(corr, keepdims=True)

    aw0 = a_x2 - a_x1
    ah0 = a_y2 - a_y1
    ctr_x = a_x1 + 0.5 * aw0
    ctr_y = a_y1 + 0.5 * ah0
    aw = jnp.where(pos, aw0, 1.0)
    ah = jnp.where(pos, ah0, 1.0)
    gt_cx = ab_x1 + 0.5 * gt_w_raw
    gt_cy = ab_y1 + 0.5 * gt_h_raw
    gt_w = jnp.maximum(gt_w_raw, 1.0)
    gt_h = jnp.maximum(gt_h_raw, 1.0)
    tdx = (gt_cx - ctr_x) / aw
    tdy = (gt_cy - ctr_y) / ah
    tdw = jnp.log(gt_w / aw)
    tdh = jnp.log(gt_h / ah)

    reg4 = reg_ref[0, 0]
    r_dy = reg4[0 * R:1 * R]
    r_dx = reg4[1 * R:2 * R]
    r_dh = reg4[2 * R:3 * R]
    r_dw = reg4[3 * R:4 * R]

    def smooth(d):
        return jnp.where(d <= 1.0 / 9.0, 4.5 * d * d, d - 1.0 / 18.0)

    rsum = (smooth(jnp.abs(tdy - r_dy)) + smooth(jnp.abs(tdx - r_dx))
            + smooth(jnp.abs(tdh - r_dh)) + smooth(jnp.abs(tdw - r_dw)))
    reg_part = jnp.sum(rsum * posf, keepdims=True)

    first = i == 0
    c0 = jnp.where(first, 0.0, acc_c[0:1, 0:1]) + cls_part
    r0 = jnp.where(first, 0.0, acc_r[0:1, 0:1]) + reg_part
    n0 = jnp.where(first, 0.0, acc_n[0:1, 0:1]) + npos_part
    acc_c[0:1, 0:1] = c0
    acc_r[0:1, 0:1] = r0
    acc_n[0:1, 0:1] = n0

    @pl.when(i == NBLK - 1)
    def _finish_batch():
        den = jnp.maximum(n0, 1.0)
        cb = c0 / den
        rb = jnp.where(n0 > 0.0, r0 / (4.0 * den), 0.0) * 50.0
        oc = jnp.where(b == 0, 0.0, oc_acc[0:1, 0:1]) + cb
        orr = jnp.where(b == 0, 0.0, or_acc[0:1, 0:1]) + rb
        oc_acc[0:1, 0:1] = oc
        or_acc[0:1, 0:1] = orr
        out_c_ref[0:1, 0:1] = oc * (1.0 / B)
        out_r_ref[0:1, 0:1] = orr * (1.0 / B)


@functools.partial(jax.jit)
def _run(tbl, anc_pack, cls, reg_pack):
    out_c, out_r = pl.pallas_call(
        _focal_kernel,
        grid=(B, NBLK),
        in_specs=[
            pl.BlockSpec(memory_space=pltpu.SMEM),
            pl.BlockSpec((1, 4 * R, 128), lambda b, i: (i, 0, 0)),
            pl.BlockSpec((1, C, BLK), lambda b, i: (b, 0, i)),
            pl.BlockSpec((1, 1, 4 * R, 128), lambda b, i: (b, i, 0, 0)),
        ],
        out_specs=[
            pl.BlockSpec((1, 1), lambda b, i: (0, 0)),
            pl.BlockSpec((1, 1), lambda b, i: (0, 0)),
        ],
        out_shape=[
            jax.ShapeDtypeStruct((1, 1), jnp.float32),
            jax.ShapeDtypeStruct((1, 1), jnp.float32),
        ],
        scratch_shapes=[pltpu.VMEM((1, 1), jnp.float32)] * 5,
        compiler_params=pltpu.CompilerParams(
            dimension_semantics=("arbitrary", "arbitrary")),
    )(tbl, anc_pack, cls, reg_pack)
    return out_c.reshape(1), out_r.reshape(1)


def kernel(detection_boxes, detection_labels, anchors, classification, regression):
    valid = detection_labels != 0
    bx = jnp.where(valid[:, :, None], detection_boxes, 1e9)
    labf = (detection_labels - 1).astype(jnp.float32)
    area_b = jnp.where(valid,
                       (bx[..., 2] - bx[..., 0]) * (bx[..., 3] - bx[..., 1]),
                       0.0)
    tbl = jnp.stack([bx[..., 0], bx[..., 1], bx[..., 2], bx[..., 3],
                     area_b, labf], axis=1)

    pad_anc = jnp.broadcast_to(
        jnp.array([-10.0, -10.0, -9.0, -9.0], jnp.float32),
        (A_PAD - A, 4))
    anc0 = jnp.concatenate([anchors[0], pad_anc], axis=0)
    anc_pack = (anc0.T.reshape(4, NBLK, R, 128)
                .transpose(1, 0, 2, 3).reshape(NBLK, 4 * R, 128))

    cls_t = jnp.transpose(classification, (0, 2, 1))

    regp = jnp.pad(regression, ((0, 0), (0, A_PAD - A), (0, 0)))
    reg_pack = (regp.transpose(0, 2, 1).reshape(B, 4, NBLK, R, 128)
                .transpose(0, 2, 1, 3, 4).reshape(B, NBLK, 4 * R, 128))

    return _run(tbl, anc_pack, cls_t, reg_pack)


# speedup vs baseline: 1.7140x; 1.0143x over previous
import functools

import jax
import jax.numpy as jnp
from jax.experimental import pallas as pl
from jax.experimental.pallas import tpu as pltpu

ALPHA = 0.25

B, A, M, C = 8, 20000, 100, 80
A_PAD = 20480
BLK = 5120
NBLK = A_PAD // BLK
R = BLK // 128


def _focal_kernel(tbl_ref, anc_ref, cls_ref, reg_ref, out_c_ref, out_r_ref,
                  acc_c, acc_r, acc_n, oc_acc, or_acc):
    b = pl.program_id(0)
    i = pl.program_id(1)

    anc = anc_ref[0]
    reg4 = reg_ref[0, 0]

    def smooth(d):
        return jnp.where(d <= 1.0 / 9.0, 4.5 * d * d, d - 1.0 / 18.0)

    def scan_half(hs, he):
        rh = he - hs
        a_y1 = anc[0 * R + hs:0 * R + he]
        a_x1 = anc[1 * R + hs:1 * R + he]
        a_y2 = anc[2 * R + hs:2 * R + he]
        a_x2 = anc[3 * R + hs:3 * R + he]
        area_a = (a_x2 - a_x1) * (a_y2 - a_y1)
        zero = jnp.zeros((rh, 128), jnp.float32)

        def body(m, carry):
            run_max, ax1, ay1, ax2, ay2, lab = carry
            bx1 = tbl_ref[b, 0, m]
            by1 = tbl_ref[b, 1, m]
            bx2 = tbl_ref[b, 2, m]
            by2 = tbl_ref[b, 3, m]
            barea = tbl_ref[b, 4, m]
            blab = tbl_ref[b, 5, m]
            iw = jnp.maximum(
                jnp.minimum(a_x2, bx2) - jnp.maximum(a_x1, bx1), 0.0)
            ih = jnp.maximum(
                jnp.minimum(a_y2, by2) - jnp.maximum(a_y1, by1), 0.0)
            inter = iw * ih
            ua = area_a + barea - inter
            iou = inter / ua
            take = iou > run_max
            return (jnp.maximum(run_max, iou),
                    jnp.where(take, bx1, ax1),
                    jnp.where(take, by1, ay1),
                    jnp.where(take, bx2, ax2),
                    jnp.where(take, by2, ay2),
                    jnp.where(take, blab, lab))

        carry = (jnp.full((rh, 128), -jnp.inf, jnp.float32),
                 zero, zero, zero, zero, zero)
        for m in range(M):
            carry = body(m, carry)
        iou_max, ab_x1, ab_y1, ab_x2, ab_y2, alab = carry

        gt_w_raw = ab_x2 - ab_x1
        gt_h_raw = ab_y2 - ab_y1
        thr = jnp.where(gt_w_raw * gt_h_raw > 100.0, 0.5, 0.15)
        pos = iou_max >= thr
        posf = jnp.where(pos, 1.0, 0.0)
        npos_h = jnp.sum(posf, keepdims=True)

        aw0 = a_x2 - a_x1
        ah0 = a_y2 - a_y1
        ctr_x = a_x1 + 0.5 * aw0
        ctr_y = a_y1 + 0.5 * ah0
        aw = jnp.where(pos, aw0, 1.0)
        ah = jnp.where(pos, ah0, 1.0)
        gt_cx = ab_x1 + 0.5 * gt_w_raw
        gt_cy = ab_y1 + 0.5 * gt_h_raw
        gt_w = jnp.maximum(gt_w_raw, 1.0)
        gt_h = jnp.maximum(gt_h_raw, 1.0)
        tdx = (gt_cx - ctr_x) / aw
        tdy = (gt_cy - ctr_y) / ah
        tdw = jnp.log(gt_w / aw)
        tdh = jnp.log(gt_h / ah)
        r_dy = reg4[0 * R + hs:0 * R + he]
        r_dx = reg4[1 * R + hs:1 * R + he]
        r_dh = reg4[2 * R + hs:2 * R + he]
        r_dw = reg4[3 * R + hs:3 * R + he]
        rsum = (smooth(jnp.abs(tdy - r_dy)) + smooth(jnp.abs(tdx - r_dx))
                + smooth(jnp.abs(tdh - r_dh)) + smooth(jnp.abs(tdw - r_dw)))
        reg_h = jnp.sum(rsum * posf, keepdims=True)

        code_h = jnp.where(pos, alab, -1.0)
        return code_h, npos_h, reg_h

    code_a, npos_a, reg_a = scan_half(0, R // 2)
    code_b, npos_b, reg_b = scan_half(R // 2, R)
    code = jnp.concatenate([code_a, code_b], axis=0)
    npos_part = npos_a + npos_b
    reg_part = reg_a + reg_b

    p = jnp.clip(cls_ref[0], 1e-4, 1.0 - 1e-4)
    neg = (0.75 * (p * p)) * (-jnp.log(1.0 - p))
    col = jax.lax.broadcasted_iota(jnp.int32, (1, BLK), 1) + i * BLK
    neg_cols = jnp.sum(neg, axis=0, keepdims=True)
    neg_sum = jnp.sum(jnp.where(col < A, neg_cols, 0.0), keepdims=True)
    code_row = code.reshape(1, BLK)
    c_iota = jax.lax.broadcasted_iota(jnp.int32, (C, 1), 0).astype(jnp.float32)
    sel = c_iota == code_row
    p_sel = jnp.sum(jnp.where(sel, p, 0.0), axis=0, keepdims=True)
    p_c = jnp.clip(p_sel, 1e-4, 1.0)
    g = (0.25 * (1.0 - p_c) * (1.0 - p_c)) * (-jnp.log(p_c)) \
        - (0.75 * (p_c * p_c)) * (-jnp.log(1.0 - p_c))
    corr = jnp.where(code_row >= 0.0, g, 0.0)
    cls_part = neg_sum + jnp.sum(corr, keepdims=True)

    first = i == 0
    c0 = jnp.where(first, 0.0, acc_c[0:1, 0:1]) + cls_part
    r0 = jnp.where(first, 0.0, acc_r[0:1, 0:1]) + reg_part
    n0 = jnp.where(first, 0.0, acc_n[0:1, 0:1]) + npos_part
    acc_c[0:1, 0:1] = c0
    acc_r[0:1, 0:1] = r0
    acc_n[0:1, 0:1] = n0

    @pl.when(i == NBLK - 1)
    def _finish_batch():
        den = jnp.maximum(n0, 1.0)
        cb = c0 / den
        rb = jnp.where(n0 > 0.0, r0 / (4.0 * den), 0.0) * 50.0
        oc = jnp.where(b == 0, 0.0, oc_acc[0:1, 0:1]) + cb
        orr = jnp.where(b == 0, 0.0, or_acc[0:1, 0:1]) + rb
        oc_acc[0:1, 0:1] = oc
        or_acc[0:1, 0:1] = orr
        out_c_ref[0:1, 0:1] = oc * (1.0 / B)
        out_r_ref[0:1, 0:1] = orr * (1.0 / B)


@functools.partial(jax.jit)
def _run(tbl, anc_pack, cls, reg_pack):
    out_c, out_r = pl.pallas_call(
        _focal_kernel,
        grid=(B, NBLK),
        in_specs=[
            pl.BlockSpec(memory_space=pltpu.SMEM),
            pl.BlockSpec((1, 4 * R, 128), lambda b, i: (i, 0, 0)),
            pl.BlockSpec((1, C, BLK), lambda b, i: (b, 0, i)),
            pl.BlockSpec((1, 1, 4 * R, 128), lambda b, i: (b, i, 0, 0)),
        ],
        out_specs=[
            pl.BlockSpec((1, 1), lambda b, i: (0, 0)),
            pl.BlockSpec((1, 1), lambda b, i: (0, 0)),
        ],
        out_shape=[
            jax.ShapeDtypeStruct((1, 1), jnp.float32),
            jax.ShapeDtypeStruct((1, 1), jnp.float32),
        ],
        scratch_shapes=[pltpu.VMEM((1, 1), jnp.float32)] * 5,
        compiler_params=pltpu.CompilerParams(
            dimension_semantics=("arbitrary", "arbitrary")),
    )(tbl, anc_pack, cls, reg_pack)
    return out_c.reshape(1), out_r.reshape(1)


def kernel(detection_boxes, detection_labels, anchors, classification, regression):
    valid = detection_labels != 0
    bx = jnp.where(valid[:, :, None], detection_boxes, 1e9)
    labf = (detection_labels - 1).astype(jnp.float32)
    area_b = jnp.where(valid,
                       (bx[..., 2] - bx[..., 0]) * (bx[..., 3] - bx[..., 1]),
                       0.0)
    tbl = jnp.stack([bx[..., 0], bx[..., 1], bx[..., 2], bx[..., 3],
                     area_b, labf], axis=1)

    pad_anc = jnp.broadcast_to(
        jnp.array([-10.0, -10.0, -9.0, -9.0], jnp.float32),
        (A_PAD - A, 4))
    anc0 = jnp.concatenate([anchors[0], pad_anc], axis=0)
    anc_pack = (anc0.T.reshape(4, NBLK, R, 128)
                .transpose(1, 0, 2, 3).reshape(NBLK, 4 * R, 128))

    cls_t = jnp.transpose(classification, (0, 2, 1))

    regp = jnp.pad(regression, ((0, 0), (0, A_PAD - A), (0, 0)))
    reg_pack = (regp.transpose(0, 2, 1).reshape(B, 4, NBLK, R, 128)
                .transpose(0, 2, 1, 3, 4).reshape(B, NBLK, 4 * R, 128))

    return _run(tbl, anc_pack, cls_t, reg_pack)


# speedup vs baseline: 1.7152x; 1.0007x over previous
import functools

import jax
import jax.numpy as jnp
from jax.experimental import pallas as pl
from jax.experimental.pallas import tpu as pltpu

ALPHA = 0.25

B, A, M, C = 8, 20000, 100, 80
A_PAD = 20480
BLK = 5120
NBLK = A_PAD // BLK
R = BLK // 128


def _focal_kernel(tbl_ref, anc_ref, cls_ref, reg_ref, out_c_ref, out_r_ref,
                  acc_c, acc_r, acc_n, oc_acc, or_acc):
    b = pl.program_id(0)
    i = pl.program_id(1)

    anc = anc_ref[0]
    reg4 = reg_ref[0, 0]

    def smooth(d):
        return jnp.where(d <= 1.0 / 9.0, 4.5 * d * d, d - 1.0 / 18.0)

    def scan_half(hs, he):
        rh = he - hs
        a_y1 = anc[0 * R + hs:0 * R + he]
        a_x1 = anc[1 * R + hs:1 * R + he]
        a_y2 = anc[2 * R + hs:2 * R + he]
        a_x2 = anc[3 * R + hs:3 * R + he]
        area_a = (a_x2 - a_x1) * (a_y2 - a_y1)
        zero = jnp.zeros((rh, 128), jnp.float32)

        def body(m, carry):
            run_max, ax1, ay1, ax2, ay2, lab = carry
            bx1 = tbl_ref[b, 0, m]
            by1 = tbl_ref[b, 1, m]
            bx2 = tbl_ref[b, 2, m]
            by2 = tbl_ref[b, 3, m]
            barea = tbl_ref[b, 4, m]
            blab = tbl_ref[b, 5, m]
            iw = jnp.maximum(
                jnp.minimum(a_x2, bx2) - jnp.maximum(a_x1, bx1), 0.0)
            ih = jnp.maximum(
                jnp.minimum(a_y2, by2) - jnp.maximum(a_y1, by1), 0.0)
            inter = iw * ih
            ua = area_a + barea - inter
            iou = inter / ua
            take = iou > run_max
            return (jnp.maximum(run_max, iou),
                    jnp.where(take, bx1, ax1),
                    jnp.where(take, by1, ay1),
                    jnp.where(take, bx2, ax2),
                    jnp.where(take, by2, ay2),
                    jnp.where(take, blab, lab))

        carry = (jnp.full((rh, 128), -jnp.inf, jnp.float32),
                 zero, zero, zero, zero, zero)
        for m in range(M):
            carry = body(m, carry)
        iou_max, ab_x1, ab_y1, ab_x2, ab_y2, alab = carry

        gt_w_raw = ab_x2 - ab_x1
        gt_h_raw = ab_y2 - ab_y1
        thr = jnp.where(gt_w_raw * gt_h_raw > 100.0, 0.5, 0.15)
        pos = iou_max >= thr
        posf = jnp.where(pos, 1.0, 0.0)
        npos_h = jnp.sum(posf, keepdims=True)

        aw0 = a_x2 - a_x1
        ah0 = a_y2 - a_y1
        ctr_x = a_x1 + 0.5 * aw0
        ctr_y = a_y1 + 0.5 * ah0
        aw = jnp.where(pos, aw0, 1.0)
        ah = jnp.where(pos, ah0, 1.0)
        gt_cx = ab_x1 + 0.5 * gt_w_raw
        gt_cy = ab_y1 + 0.5 * gt_h_raw
        gt_w = jnp.maximum(gt_w_raw, 1.0)
        gt_h = jnp.maximum(gt_h_raw, 1.0)
        tdx = (gt_cx - ctr_x) / aw
        tdy = (gt_cy - ctr_y) / ah
        tdw = jnp.log(gt_w / aw)
        tdh = jnp.log(gt_h / ah)
        r_dy = reg4[0 * R + hs:0 * R + he]
        r_dx = reg4[1 * R + hs:1 * R + he]
        r_dh = reg4[2 * R + hs:2 * R + he]
        r_dw = reg4[3 * R + hs:3 * R + he]
        rsum = (smooth(jnp.abs(tdy - r_dy)) + smooth(jnp.abs(tdx - r_dx))
                + smooth(jnp.abs(tdh - r_dh)) + smooth(jnp.abs(tdw - r_dw)))
        reg_h = jnp.sum(rsum * posf, keepdims=True)

        code_h = jnp.where(pos, alab, -1.0)
        return code_h, npos_h, reg_h

    parts = [scan_half(h * (R // 4), (h + 1) * (R // 4)) for h in range(4)]
    code = jnp.concatenate([ph[0] for ph in parts], axis=0)
    npos_part = sum(ph[1] for ph in parts)
    reg_part = sum(ph[2] for ph in parts)

    p = jnp.clip(cls_ref[0], 1e-4, 1.0 - 1e-4)
    neg = (0.75 * (p * p)) * (-jnp.log(1.0 - p))
    col = jax.lax.broadcasted_iota(jnp.int32, (1, BLK), 1) + i * BLK
    neg_cols = jnp.sum(neg, axis=0, keepdims=True)
    neg_sum = jnp.sum(jnp.where(col < A, neg_cols, 0.0), keepdims=True)
    code_row = code.reshape(1, BLK)
    c_iota = jax.lax.broadcasted_iota(jnp.int32, (C, 1), 0).astype(jnp.float32)
    sel = c_iota == code_row
    p_sel = jnp.sum(jnp.where(sel, p, 0.0), axis=0, keepdims=True)
    p_c = jnp.clip(p_sel, 1e-4, 1.0)
    g = (0.25 * (1.0 - p_c) * (1.0 - p_c)) * (-jnp.log(p_c)) \
        - (0.75 * (p_c * p_c)) * (-jnp.log(1.0 - p_c))
    corr = jnp.where(code_row >= 0.0, g, 0.0)
    cls_part = neg_sum + jnp.sum(corr, keepdims=True)

    first = i == 0
    c0 = jnp.where(first, 0.0, acc_c[0:1, 0:1]) + cls_part
    r0 = jnp.where(first, 0.0, acc_r[0:1, 0:1]) + reg_part
    n0 = jnp.where(first, 0.0, acc_n[0:1, 0:1]) + npos_part
    acc_c[0:1, 0:1] = c0
    acc_r[0:1, 0:1] = r0
    acc_n[0:1, 0:1] = n0

    @pl.when(i == NBLK - 1)
    def _finish_batch():
        den = jnp.maximum(n0, 1.0)
        cb = c0 / den
        rb = jnp.where(n0 > 0.0, r0 / (4.0 * den), 0.0) * 50.0
        oc = jnp.where(b == 0, 0.0, oc_acc[0:1, 0:1]) + cb
        orr = jnp.where(b == 0, 0.0, or_acc[0:1, 0:1]) + rb
        oc_acc[0:1, 0:1] = oc
        or_acc[0:1, 0:1] = orr
        out_c_ref[0:1, 0:1] = oc * (1.0 / B)
        out_r_ref[0:1, 0:1] = orr * (1.0 / B)


@functools.partial(jax.jit)
def _run(tbl, anc_pack, cls, reg_pack):
    out_c, out_r = pl.pallas_call(
        _focal_kernel,
        grid=(B, NBLK),
        in_specs=[
            pl.BlockSpec(memory_space=pltpu.SMEM),
            pl.BlockSpec((1, 4 * R, 128), lambda b, i: (i, 0, 0)),
            pl.BlockSpec((1, C, BLK), lambda b, i: (b, 0, i)),
            pl.BlockSpec((1, 1, 4 * R, 128), lambda b, i: (b, i, 0, 0)),
        ],
        out_specs=[
            pl.BlockSpec((1, 1), lambda b, i: (0, 0)),
            pl.BlockSpec((1, 1), lambda b, i: (0, 0)),
        ],
        out_shape=[
            jax.ShapeDtypeStruct((1, 1), jnp.float32),
            jax.ShapeDtypeStruct((1, 1), jnp.float32),
        ],
        scratch_shapes=[pltpu.VMEM((1, 1), jnp.float32)] * 5,
        compiler_params=pltpu.CompilerParams(
            dimension_semantics=("arbitrary", "arbitrary")),
    )(tbl, anc_pack, cls, reg_pack)
    return out_c.reshape(1), out_r.reshape(1)


def kernel(detection_boxes, detection_labels, anchors, classification, regression):
    valid = detection_labels != 0
    bx = jnp.where(valid[:, :, None], detection_boxes, 1e9)
    labf = (detection_labels - 1).astype(jnp.float32)
    area_b = jnp.where(valid,
                       (bx[..., 2] - bx[..., 0]) * (bx[..., 3] - bx[..., 1]),
                       0.0)
    tbl = jnp.stack([bx[..., 0], bx[..., 1], bx[..., 2], bx[..., 3],
                     area_b, labf], axis=1)

    pad_anc = jnp.broadcast_to(
        jnp.array([-10.0, -10.0, -9.0, -9.0], jnp.float32),
        (A_PAD - A, 4))
    anc0 = jnp.concatenate([anchors[0], pad_anc], axis=0)
    anc_pack = (anc0.T.reshape(4, NBLK, R, 128)
                .transpose(1, 0, 2, 3).reshape(NBLK, 4 * R, 128))

    cls_t = jnp.transpose(classification, (0, 2, 1))

    regp = jnp.pad(regression, ((0, 0), (0, A_PAD - A), (0, 0)))
    reg_pack = (regp.transpose(0, 2, 1).reshape(B, 4, NBLK, R, 128)
                .transpose(0, 2, 1, 3, 4).reshape(B, NBLK, 4 * R, 128))

    return _run(tbl, anc_pack, cls_t, reg_pack)


# speedup vs baseline: 1.7244x; 1.0054x over previous
import functools

import jax
import jax.numpy as jnp
from jax.experimental import pallas as pl
from jax.experimental.pallas import tpu as pltpu

ALPHA = 0.25

B, A, M, C = 8, 20000, 100, 80
A_PAD = 20480
BLK = 10240
NBLK = A_PAD // BLK
R = BLK // 128


def _focal_kernel(tbl_ref, anc_ref, cls_ref, reg_ref, out_c_ref, out_r_ref,
                  acc_c, acc_r, acc_n, oc_acc, or_acc):
    b = pl.program_id(0)
    i = pl.program_id(1)

    anc = anc_ref[0]
    reg4 = reg_ref[0, 0]

    def smooth(d):
        return jnp.where(d <= 1.0 / 9.0, 4.5 * d * d, d - 1.0 / 18.0)

    def scan_half(hs, he):
        rh = he - hs
        a_y1 = anc[0 * R + hs:0 * R + he]
        a_x1 = anc[1 * R + hs:1 * R + he]
        a_y2 = anc[2 * R + hs:2 * R + he]
        a_x2 = anc[3 * R + hs:3 * R + he]
        area_a = (a_x2 - a_x1) * (a_y2 - a_y1)
        zero = jnp.zeros((rh, 128), jnp.float32)

        def body(m, carry):
            run_max, ax1, ay1, ax2, ay2, lab = carry
            bx1 = tbl_ref[b, 0, m]
            by1 = tbl_ref[b, 1, m]
            bx2 = tbl_ref[b, 2, m]
            by2 = tbl_ref[b, 3, m]
            barea = tbl_ref[b, 4, m]
            blab = tbl_ref[b, 5, m]
            iw = jnp.maximum(
                jnp.minimum(a_x2, bx2) - jnp.maximum(a_x1, bx1), 0.0)
            ih = jnp.maximum(
                jnp.minimum(a_y2, by2) - jnp.maximum(a_y1, by1), 0.0)
            inter = iw * ih
            ua = area_a + barea - inter
            iou = inter / ua
            take = iou > run_max
            return (jnp.maximum(run_max, iou),
                    jnp.where(take, bx1, ax1),
                    jnp.where(take, by1, ay1),
                    jnp.where(take, bx2, ax2),
                    jnp.where(take, by2, ay2),
                    jnp.where(take, blab, lab))

        carry = (jnp.full((rh, 128), -jnp.inf, jnp.float32),
                 zero, zero, zero, zero, zero)
        for m in range(M):
            carry = body(m, carry)
        iou_max, ab_x1, ab_y1, ab_x2, ab_y2, alab = carry

        gt_w_raw = ab_x2 - ab_x1
        gt_h_raw = ab_y2 - ab_y1
        thr = jnp.where(gt_w_raw * gt_h_raw > 100.0, 0.5, 0.15)
        pos = iou_max >= thr
        posf = jnp.where(pos, 1.0, 0.0)
        npos_h = jnp.sum(posf, keepdims=True)

        aw0 = a_x2 - a_x1
        ah0 = a_y2 - a_y1
        ctr_x = a_x1 + 0.5 * aw0
        ctr_y = a_y1 + 0.5 * ah0
        aw = jnp.where(pos, aw0, 1.0)
        ah = jnp.where(pos, ah0, 1.0)
        gt_cx = ab_x1 + 0.5 * gt_w_raw
        gt_cy = ab_y1 + 0.5 * gt_h_raw
        gt_w = jnp.maximum(gt_w_raw, 1.0)
        gt_h = jnp.maximum(gt_h_raw, 1.0)
        tdx = (gt_cx - ctr_x) / aw
        tdy = (gt_cy - ctr_y) / ah
        tdw = jnp.log(gt_w / aw)
        tdh = jnp.log(gt_h / ah)
        r_dy = reg4[0 * R + hs:0 * R + he]
        r_dx = reg4[1 * R + hs:1 * R + he]
        r_dh = reg4[2 * R + hs:2 * R + he]
        r_dw = reg4[3 * R + hs:3 * R + he]
        rsum = (smooth(jnp.abs(tdy - r_dy)) + smooth(jnp.abs(tdx - r_dx))
                + smooth(jnp.abs(tdh - r_dh)) + smooth(jnp.abs(tdw - r_dw)))
        reg_h = jnp.sum(rsum * posf, keepdims=True)

        code_h = jnp.where(pos, alab, -1.0)
        return code_h, npos_h, reg_h

    parts = [scan_half(h * (R // 4), (h + 1) * (R // 4)) for h in range(4)]
    code = jnp.concatenate([ph[0] for ph in parts], axis=0)
    npos_part = sum(ph[1] for ph in parts)
    reg_part = sum(ph[2] for ph in parts)

    p = jnp.clip(cls_ref[0], 1e-4, 1.0 - 1e-4)
    neg = (0.75 * (p * p)) * (-jnp.log(1.0 - p))
    col = jax.lax.broadcasted_iota(jnp.int32, (1, BLK), 1) + i * BLK
    neg_cols = jnp.sum(neg, axis=0, keepdims=True)
    neg_sum = jnp.sum(jnp.where(col < A, neg_cols, 0.0), keepdims=True)
    code_row = code.reshape(1, BLK)
    c_iota = jax.lax.broadcasted_iota(jnp.int32, (C, 1), 0).astype(jnp.float32)
    sel = c_iota == code_row
    p_sel = jnp.sum(jnp.where(sel, p, 0.0), axis=0, keepdims=True)
    p_c = jnp.clip(p_sel, 1e-4, 1.0)
    g = (0.25 * (1.0 - p_c) * (1.0 - p_c)) * (-jnp.log(p_c)) \
        - (0.75 * (p_c * p_c)) * (-jnp.log(1.0 - p_c))
    corr = jnp.where(code_row >= 0.0, g, 0.0)
    cls_part = neg_sum + jnp.sum(corr, keepdims=True)

    first = i == 0
    c0 = jnp.where(first, 0.0, acc_c[0:1, 0:1]) + cls_part
    r0 = jnp.where(first, 0.0, acc_r[0:1, 0:1]) + reg_part
    n0 = jnp.where(first, 0.0, acc_n[0:1, 0:1]) + npos_part
    acc_c[0:1, 0:1] = c0
    acc_r[0:1, 0:1] = r0
    acc_n[0:1, 0:1] = n0

    @pl.when(i == NBLK - 1)
    def _finish_batch():
        den = jnp.maximum(n0, 1.0)
        cb = c0 / den
        rb = jnp.where(n0 > 0.0, r0 / (4.0 * den), 0.0) * 50.0
        oc = jnp.where(b == 0, 0.0, oc_acc[0:1, 0:1]) + cb
        orr = jnp.where(b == 0, 0.0, or_acc[0:1, 0:1]) + rb
        oc_acc[0:1, 0:1] = oc
        or_acc[0:1, 0:1] = orr
        out_c_ref[0:1, 0:1] = oc * (1.0 / B)
        out_r_ref[0:1, 0:1] = orr * (1.0 / B)


@functools.partial(jax.jit)
def _run(tbl, anc_pack, cls, reg_pack):
    out_c, out_r = pl.pallas_call(
        _focal_kernel,
        grid=(B, NBLK),
        in_specs=[
            pl.BlockSpec(memory_space=pltpu.SMEM),
            pl.BlockSpec((1, 4 * R, 128), lambda b, i: (i, 0, 0)),
            pl.BlockSpec((1, C, BLK), lambda b, i: (b, 0, i)),
            pl.BlockSpec((1, 1, 4 * R, 128), lambda b, i: (b, i, 0, 0)),
        ],
        out_specs=[
            pl.BlockSpec((1, 1), lambda b, i: (0, 0)),
            pl.BlockSpec((1, 1), lambda b, i: (0, 0)),
        ],
        out_shape=[
            jax.ShapeDtypeStruct((1, 1), jnp.float32),
            jax.ShapeDtypeStruct((1, 1), jnp.float32),
        ],
        scratch_shapes=[pltpu.VMEM((1, 1), jnp.float32)] * 5,
        compiler_params=pltpu.CompilerParams(
            dimension_semantics=("arbitrary", "arbitrary")),
    )(tbl, anc_pack, cls, reg_pack)
    return out_c.reshape(1), out_r.reshape(1)


def kernel(detection_boxes, detection_labels, anchors, classification, regression):
    valid = detection_labels != 0
    bx = jnp.where(valid[:, :, None], detection_boxes, 1e9)
    labf = (detection_labels - 1).astype(jnp.float32)
    area_b = jnp.where(valid,
                       (bx[..., 2] - bx[..., 0]) * (bx[..., 3] - bx[..., 1]),
                       0.0)
    tbl = jnp.stack([bx[..., 0], bx[..., 1], bx[..., 2], bx[..., 3],
                     area_b, labf], axis=1)

    pad_anc = jnp.broadcast_to(
        jnp.array([-10.0, -10.0, -9.0, -9.0], jnp.float32),
        (A_PAD - A, 4))
    anc0 = jnp.concatenate([anchors[0], pad_anc], axis=0)
    anc_pack = (anc0.T.reshape(4, NBLK, R, 128)
                .transpose(1, 0, 2, 3).reshape(NBLK, 4 * R, 128))

    cls_t = jnp.transpose(classification, (0, 2, 1))

    regp = jnp.pad(regression, ((0, 0), (0, A_PAD - A), (0, 0)))
    reg_pack = (regp.transpose(0, 2, 1).reshape(B, 4, NBLK, R, 128)
                .transpose(0, 2, 1, 3, 4).reshape(B, NBLK, 4 * R, 128))

    return _run(tbl, anc_pack, cls_t, reg_pack)


# speedup vs baseline: 1.7384x; 1.0081x over previous
import functools

import jax
import jax.numpy as jnp
from jax.experimental import pallas as pl
from jax.experimental.pallas import tpu as pltpu

ALPHA = 0.25

B, A, M, C = 8, 20000, 100, 80
A_PAD = 20480
BLK = 20480
NBLK = A_PAD // BLK
R = BLK // 128


def _focal_kernel(tbl_ref, anc_ref, cls_ref, reg_ref, out_c_ref, out_r_ref,
                  acc_c, acc_r, acc_n, oc_acc, or_acc):
    b = pl.program_id(0)
    i = pl.program_id(1)

    anc = anc_ref[0]
    reg4 = reg_ref[0, 0]

    def smooth(d):
        return jnp.where(d <= 1.0 / 9.0, 4.5 * d * d, d - 1.0 / 18.0)

    def scan_half(hs, he):
        rh = he - hs
        a_y1 = anc[0 * R + hs:0 * R + he]
        a_x1 = anc[1 * R + hs:1 * R + he]
        a_y2 = anc[2 * R + hs:2 * R + he]
        a_x2 = anc[3 * R + hs:3 * R + he]
        area_a = (a_x2 - a_x1) * (a_y2 - a_y1)
        zero = jnp.zeros((rh, 128), jnp.float32)

        def body(m, carry):
            run_max, ax1, ay1, ax2, ay2, lab = carry
            bx1 = tbl_ref[b, 0, m]
            by1 = tbl_ref[b, 1, m]
            bx2 = tbl_ref[b, 2, m]
            by2 = tbl_ref[b, 3, m]
            barea = tbl_ref[b, 4, m]
            blab = tbl_ref[b, 5, m]
            iw = jnp.maximum(
                jnp.minimum(a_x2, bx2) - jnp.maximum(a_x1, bx1), 0.0)
            ih = jnp.maximum(
                jnp.minimum(a_y2, by2) - jnp.maximum(a_y1, by1), 0.0)
            inter = iw * ih
            ua = area_a + barea - inter
            iou = inter / ua
            take = iou > run_max
            return (jnp.maximum(run_max, iou),
                    jnp.where(take, bx1, ax1),
                    jnp.where(take, by1, ay1),
                    jnp.where(take, bx2, ax2),
                    jnp.where(take, by2, ay2),
                    jnp.where(take, blab, lab))

        carry = (jnp.full((rh, 128), -jnp.inf, jnp.float32),
                 zero, zero, zero, zero, zero)
        for m in range(M):
            carry = body(m, carry)
        iou_max, ab_x1, ab_y1, ab_x2, ab_y2, alab = carry

        gt_w_raw = ab_x2 - ab_x1
        gt_h_raw = ab_y2 - ab_y1
        thr = jnp.where(gt_w_raw * gt_h_raw > 100.0, 0.5, 0.15)
        pos = iou_max >= thr
        posf = jnp.where(pos, 1.0, 0.0)
        npos_h = jnp.sum(posf, keepdims=True)

        aw0 = a_x2 - a_x1
        ah0 = a_y2 - a_y1
        ctr_x = a_x1 + 0.5 * aw0
        ctr_y = a_y1 + 0.5 * ah0
        aw = jnp.where(pos, aw0, 1.0)
        ah = jnp.where(pos, ah0, 1.0)
        gt_cx = ab_x1 + 0.5 * gt_w_raw
        gt_cy = ab_y1 + 0.5 * gt_h_raw
        gt_w = jnp.maximum(gt_w_raw, 1.0)
        gt_h = jnp.maximum(gt_h_raw, 1.0)
        tdx = (gt_cx - ctr_x) / aw
        tdy = (gt_cy - ctr_y) / ah
        tdw = jnp.log(gt_w / aw)
        tdh = jnp.log(gt_h / ah)
        r_dy = reg4[0 * R + hs:0 * R + he]
        r_dx = reg4[1 * R + hs:1 * R + he]
        r_dh = reg4[2 * R + hs:2 * R + he]
        r_dw = reg4[3 * R + hs:3 * R + he]
        rsum = (smooth(jnp.abs(tdy - r_dy)) + smooth(jnp.abs(tdx - r_dx))
                + smooth(jnp.abs(tdh - r_dh)) + smooth(jnp.abs(tdw - r_dw)))
        reg_h = jnp.sum(rsum * posf, keepdims=True)

        code_h = jnp.where(pos, alab, -1.0)
        return code_h, npos_h, reg_h

    parts = [scan_half(h * (R // 8), (h + 1) * (R // 8)) for h in range(8)]
    code = jnp.concatenate([ph[0] for ph in parts], axis=0)
    npos_part = sum(ph[1] for ph in parts)
    reg_part = sum(ph[2] for ph in parts)

    p = jnp.clip(cls_ref[0], 1e-4, 1.0 - 1e-4)
    neg = (0.75 * (p * p)) * (-jnp.log(1.0 - p))
    col = jax.lax.broadcasted_iota(jnp.int32, (1, BLK), 1) + i * BLK
    neg_cols = jnp.sum(neg, axis=0, keepdims=True)
    neg_sum = jnp.sum(jnp.where(col < A, neg_cols, 0.0), keepdims=True)
    code_row = code.reshape(1, BLK)
    c_iota = jax.lax.broadcasted_iota(jnp.int32, (C, 1), 0).astype(jnp.float32)
    sel = c_iota == code_row
    p_sel = jnp.sum(jnp.where(sel, p, 0.0), axis=0, keepdims=True)
    p_c = jnp.clip(p_sel, 1e-4, 1.0)
    g = (0.25 * (1.0 - p_c) * (1.0 - p_c)) * (-jnp.log(p_c)) \
        - (0.75 * (p_c * p_c)) * (-jnp.log(1.0 - p_c))
    corr = jnp.where(code_row >= 0.0, g, 0.0)
    cls_part = neg_sum + jnp.sum(corr, keepdims=True)

    first = i == 0
    c0 = jnp.where(first, 0.0, acc_c[0:1, 0:1]) + cls_part
    r0 = jnp.where(first, 0.0, acc_r[0:1, 0:1]) + reg_part
    n0 = jnp.where(first, 0.0, acc_n[0:1, 0:1]) + npos_part
    acc_c[0:1, 0:1] = c0
    acc_r[0:1, 0:1] = r0
    acc_n[0:1, 0:1] = n0

    @pl.when(i == NBLK - 1)
    def _finish_batch():
        den = jnp.maximum(n0, 1.0)
        cb = c0 / den
        rb = jnp.where(n0 > 0.0, r0 / (4.0 * den), 0.0) * 50.0
        oc = jnp.where(b == 0, 0.0, oc_acc[0:1, 0:1]) + cb
        orr = jnp.where(b == 0, 0.0, or_acc[0:1, 0:1]) + rb
        oc_acc[0:1, 0:1] = oc
        or_acc[0:1, 0:1] = orr
        out_c_ref[0:1, 0:1] = oc * (1.0 / B)
        out_r_ref[0:1, 0:1] = orr * (1.0 / B)


@functools.partial(jax.jit)
def _run(tbl, anc_pack, cls, reg_pack):
    out_c, out_r = pl.pallas_call(
        _focal_kernel,
        grid=(B, NBLK),
        in_specs=[
            pl.BlockSpec(memory_space=pltpu.SMEM),
            pl.BlockSpec((1, 4 * R, 128), lambda b, i: (i, 0, 0)),
            pl.BlockSpec((1, C, BLK), lambda b, i: (b, 0, i)),
            pl.BlockSpec((1, 1, 4 * R, 128), lambda b, i: (b, i, 0, 0)),
        ],
        out_specs=[
            pl.BlockSpec((1, 1), lambda b, i: (0, 0)),
            pl.BlockSpec((1, 1), lambda b, i: (0, 0)),
        ],
        out_shape=[
            jax.ShapeDtypeStruct((1, 1), jnp.float32),
            jax.ShapeDtypeStruct((1, 1), jnp.float32),
        ],
        scratch_shapes=[pltpu.VMEM((1, 1), jnp.float32)] * 5,
        compiler_params=pltpu.CompilerParams(
            dimension_semantics=("arbitrary", "arbitrary")),
    )(tbl, anc_pack, cls, reg_pack)
    return out_c.reshape(1), out_r.reshape(1)


def kernel(detection_boxes, detection_labels, anchors, classification, regression):
    valid = detection_labels != 0
    bx = jnp.where(valid[:, :, None], detection_boxes, 1e9)
    labf = (detection_labels - 1).astype(jnp.float32)
    area_b = jnp.where(valid,
                       (bx[..., 2] - bx[..., 0]) * (bx[..., 3] - bx[..., 1]),
                       0.0)
    tbl = jnp.stack([bx[..., 0], bx[..., 1], bx[..., 2], bx[..., 3],
                     area_b, labf], axis=1)

    pad_anc = jnp.broadcast_to(
        jnp.array([-10.0, -10.0, -9.0, -9.0], jnp.float32),
        (A_PAD - A, 4))
    anc0 = jnp.concatenate([anchors[0], pad_anc], axis=0)
    anc_pack = (anc0.T.reshape(4, NBLK, R, 128)
                .transpose(1, 0, 2, 3).reshape(NBLK, 4 * R, 128))

    cls_t = jnp.transpose(classification, (0, 2, 1))

    regp = jnp.pad(regression, ((0, 0), (0, A_PAD - A), (0, 0)))
    reg_pack = (regp.transpose(0, 2, 1).reshape(B, 4, NBLK, R, 128)
                .transpose(0, 2, 1, 3, 4).reshape(B, NBLK, 4 * R, 128))

    return _run(tbl, anc_pack, cls_t, reg_pack)


# speedup vs baseline: 1.7407x; 1.0013x over previous
import functools

import jax
import jax.numpy as jnp
from jax.experimental import pallas as pl
from jax.experimental.pallas import tpu as pltpu

B, A, M, C = 8, 20000, 100, 80
A_PAD = 20480
BLK = 20480
NBLK = A_PAD // BLK
R = BLK // 128


def _focal_kernel(tbl_ref, anc_ref, cls_ref, reg_ref, out_c_ref, out_r_ref,
                  acc_c, acc_r, acc_n, oc_acc, or_acc):
    b = pl.program_id(0)
    i = pl.program_id(1)

    anc = anc_ref[0]
    reg4 = reg_ref[0, 0]

    def smooth(d):
        return jnp.where(d <= 1.0 / 9.0, 4.5 * d * d, d - 1.0 / 18.0)

    def scan_half(hs, he):
        rh = he - hs
        a_y1 = anc[0 * R + hs:0 * R + he]
        a_x1 = anc[1 * R + hs:1 * R + he]
        a_y2 = anc[2 * R + hs:2 * R + he]
        a_x2 = anc[3 * R + hs:3 * R + he]
        area_a = (a_x2 - a_x1) * (a_y2 - a_y1)
        zero = jnp.zeros((rh, 128), jnp.float32)

        def body(m, carry):
            run_max, ax1, ay1, ax2, ay2, lab = carry
            bx1 = tbl_ref[b, 0, m]
            by1 = tbl_ref[b, 1, m]
            bx2 = tbl_ref[b, 2, m]
            by2 = tbl_ref[b, 3, m]
            barea = tbl_ref[b, 4, m]
            blab = tbl_ref[b, 5, m]
            iw = jnp.maximum(
                jnp.minimum(a_x2, bx2) - jnp.maximum(a_x1, bx1), 0.0)
            ih = jnp.maximum(
                jnp.minimum(a_y2, by2) - jnp.maximum(a_y1, by1), 0.0)
            inter = iw * ih
            ua = area_a + barea - inter
            iou = inter / ua
            take = iou > run_max
            return (jnp.maximum(run_max, iou),
                    jnp.where(take, bx1, ax1),
                    jnp.where(take, by1, ay1),
                    jnp.where(take, bx2, ax2),
                    jnp.where(take, by2, ay2),
                    jnp.where(take, blab, lab))

        carry = (jnp.full((rh, 128), -jnp.inf, jnp.float32),
                 zero, zero, zero, zero, zero)
        for m in range(M):
            carry = body(m, carry)
        iou_max, ab_x1, ab_y1, ab_x2, ab_y2, alab = carry

        gt_w_raw = ab_x2 - ab_x1
        gt_h_raw = ab_y2 - ab_y1
        thr = jnp.where(gt_w_raw * gt_h_raw > 100.0, 0.5, 0.15)
        pos = iou_max >= thr
        posf = jnp.where(pos, 1.0, 0.0)
        npos_h = jnp.sum(posf, keepdims=True)

        aw0 = a_x2 - a_x1
        ah0 = a_y2 - a_y1
        ctr_x = a_x1 + 0.5 * aw0
        ctr_y = a_y1 + 0.5 * ah0
        aw = jnp.where(pos, aw0, 1.0)
        ah = jnp.where(pos, ah0, 1.0)
        gt_cx = ab_x1 + 0.5 * gt_w_raw
        gt_cy = ab_y1 + 0.5 * gt_h_raw
        gt_w = jnp.maximum(gt_w_raw, 1.0)
        gt_h = jnp.maximum(gt_h_raw, 1.0)
        tdx = (gt_cx - ctr_x) / aw
        tdy = (gt_cy - ctr_y) / ah
        tdw = jnp.log(gt_w / aw)
        tdh = jnp.log(gt_h / ah)
        r_dy = reg4[0 * R + hs:0 * R + he]
        r_dx = reg4[1 * R + hs:1 * R + he]
        r_dh = reg4[2 * R + hs:2 * R + he]
        r_dw = reg4[3 * R + hs:3 * R + he]
        rsum = (smooth(jnp.abs(tdy - r_dy)) + smooth(jnp.abs(tdx - r_dx))
                + smooth(jnp.abs(tdh - r_dh)) + smooth(jnp.abs(tdw - r_dw)))
        reg_h = jnp.sum(rsum * posf, keepdims=True)

        code_h = jnp.where(pos, alab, -1.0)
        return code_h, npos_h, reg_h

    parts = [scan_half(h * (R // 8), (h + 1) * (R // 8)) for h in range(8)]
    code = jnp.concatenate([ph[0] for ph in parts], axis=0)
    npos_part = sum(ph[1] for ph in parts)
    reg_part = sum(ph[2] for ph in parts)

    p = jnp.clip(cls_ref[0], 1e-4, 1.0 - 1e-4)
    neg = (0.75 * (p * p)) * (-jnp.log(1.0 - p))
    col = jax.lax.broadcasted_iota(jnp.int32, (1, BLK), 1) + i * BLK
    neg_cols = jnp.sum(neg, axis=0, keepdims=True)
    neg_sum = jnp.sum(jnp.where(col < A, neg_cols, 0.0), keepdims=True)
    code_row = code.reshape(1, BLK)
    c_iota = jax.lax.broadcasted_iota(jnp.int32, (C, 1), 0).astype(jnp.float32)
    sel = c_iota == code_row
    p_sel = jnp.sum(jnp.where(sel, p, 0.0), axis=0, keepdims=True)
    p_c = jnp.clip(p_sel, 1e-4, 1.0)
    g = (0.25 * (1.0 - p_c) * (1.0 - p_c)) * (-jnp.log(p_c)) \
        - (0.75 * (p_c * p_c)) * (-jnp.log(1.0 - p_c))
    corr = jnp.where(code_row >= 0.0, g, 0.0)
    cls_part = neg_sum + jnp.sum(corr, keepdims=True)

    first = i == 0
    c0 = jnp.where(first, 0.0, acc_c[0:1, 0:1]) + cls_part
    r0 = jnp.where(first, 0.0, acc_r[0:1, 0:1]) + reg_part
    n0 = jnp.where(first, 0.0, acc_n[0:1, 0:1]) + npos_part
    acc_c[0:1, 0:1] = c0
    acc_r[0:1, 0:1] = r0
    acc_n[0:1, 0:1] = n0

    @pl.when(i == NBLK - 1)
    def _finish_batch():
        den = jnp.maximum(n0, 1.0)
        cb = c0 / den
        rb = jnp.where(n0 > 0.0, r0 / (4.0 * den), 0.0) * 50.0
        oc = jnp.where(b == 0, 0.0, oc_acc[0:1, 0:1]) + cb
        orr = jnp.where(b == 0, 0.0, or_acc[0:1, 0:1]) + rb
        oc_acc[0:1, 0:1] = oc
        or_acc[0:1, 0:1] = orr
        out_c_ref[0:1, 0:1] = oc * (1.0 / B)
        out_r_ref[0:1, 0:1] = orr * (1.0 / B)


@functools.partial(jax.jit)
def _run(tbl, anc_pack, cls, reg_pack):
    out_c, out_r = pl.pallas_call(
        _focal_kernel,
        grid=(B, NBLK),
        in_specs=[
            pl.BlockSpec(memory_space=pltpu.SMEM),
            pl.BlockSpec((1, 4 * R, 128), lambda b, i: (i, 0, 0)),
            pl.BlockSpec((1, C, BLK), lambda b, i: (b, 0, i)),
            pl.BlockSpec((1, 1, 4 * R, 128), lambda b, i: (b, i, 0, 0)),
        ],
        out_specs=[
            pl.BlockSpec((1, 1), lambda b, i: (0, 0)),
            pl.BlockSpec((1, 1), lambda b, i: (0, 0)),
        ],
        out_shape=[
            jax.ShapeDtypeStruct((1, 1), jnp.float32),
            jax.ShapeDtypeStruct((1, 1), jnp.float32),
        ],
        scratch_shapes=[pltpu.VMEM((1, 1), jnp.float32)] * 5,
        compiler_params=pltpu.CompilerParams(
            dimension_semantics=("arbitrary", "arbitrary")),
    )(tbl, anc_pack, cls, reg_pack)
    return out_c.reshape(1), out_r.reshape(1)


def kernel(detection_boxes, detection_labels, anchors, classification, regression):
    valid = detection_labels != 0
    bx = jnp.where(valid[:, :, None], detection_boxes, 1e9)
    labf = (detection_labels - 1).astype(jnp.float32)
    area_b = jnp.where(valid,
                       (bx[..., 2] - bx[..., 0]) * (bx[..., 3] - bx[..., 1]),
                       0.0)
    tbl = jnp.stack([bx[..., 0], bx[..., 1], bx[..., 2], bx[..., 3],
                     area_b, labf], axis=1)

    pad_anc = jnp.broadcast_to(
        jnp.array([-10.0, -10.0, -9.0, -9.0], jnp.float32),
        (A_PAD - A, 4))
    anc0 = jnp.concatenate([anchors[0], pad_anc], axis=0)
    anc_pack = (anc0.T.reshape(4, NBLK, R, 128)
                .transpose(1, 0, 2, 3).reshape(NBLK, 4 * R, 128))

    cls_t = jnp.transpose(classification, (0, 2, 1))

    regp = jnp.pad(regression, ((0, 0), (0, A_PAD - A), (0, 0)))
    reg_pack = (regp.transpose(0, 2, 1).reshape(B, 4, NBLK, R, 128)
                .transpose(0, 2, 1, 3, 4).reshape(B, NBLK, 4 * R, 128))

    return _run(tbl, anc_pack, cls_t, reg_pack)
